# Initial kernel scaffold; baseline (speedup 1.0000x reference)
import jax
import jax.numpy as jnp
from jax.experimental import pallas as pl


def kernel(x):
    raise NotImplementedError("write your pallas kernel here")


# speedup vs baseline: 3.9415x; 3.9415x over previous
import jax
import jax.numpy as jnp
from jax.experimental import pallas as pl
from jax.experimental.pallas import tpu as pltpu

_NUM = 16


def _copy_body(idx_ref, in_ref, out_ref):
    out_ref[...] = in_ref[...]


def kernel(x):
    b, c, t, h, w = x.shape
    idx = jnp.clip(jnp.linspace(0.0, t - 1, _NUM), 0, t - 1).astype(jnp.int32)
    bc = b * c
    xr = x.reshape(bc, t, h, w)
    out = pl.pallas_call(
        _copy_body,
        grid_spec=pltpu.PrefetchScalarGridSpec(
            num_scalar_prefetch=1,
            grid=(bc, _NUM),
            in_specs=[
                pl.BlockSpec((1, 1, h, w), lambda i, j, idx_ref: (i, idx_ref[j], 0, 0)),
            ],
            out_specs=pl.BlockSpec((1, 1, h, w), lambda i, j, idx_ref: (i, j, 0, 0)),
        ),
        out_shape=jax.ShapeDtypeStruct((bc, _NUM, h, w), x.dtype),
    )(idx, xr)
    return out.reshape(b, c, _NUM, h, w)


# speedup vs baseline: 15.3046x; 3.8829x over previous
import jax
import jax.numpy as jnp
from jax.experimental import pallas as pl
from jax.experimental.pallas import tpu as pltpu

_NUM = 16


def _copy_body(idx_ref, in_ref, out_ref):
    out_ref[...] = in_ref[...]


def kernel(x):
    b, c, t, h, w = x.shape
    idx = jnp.clip(jnp.linspace(0.0, t - 1, _NUM), 0, t - 1).astype(jnp.int32)
    bc = b * c
    xr = x.reshape(bc, t, h, w)
    out = pl.pallas_call(
        _copy_body,
        grid_spec=pltpu.PrefetchScalarGridSpec(
            num_scalar_prefetch=1,
            grid=(_NUM,),
            in_specs=[
                pl.BlockSpec((bc, 1, h, w), lambda j, idx_ref: (0, idx_ref[j], 0, 0)),
            ],
            out_specs=pl.BlockSpec((bc, 1, h, w), lambda j, idx_ref: (0, j, 0, 0)),
        ),
        out_shape=jax.ShapeDtypeStruct((bc, _NUM, h, w), x.dtype),
    )(idx, xr)
    return out.reshape(b, c, _NUM, h, w)
